# baseline (speedup 1.0000x reference)
import functools

import jax
import jax.numpy as jnp
from jax import lax
from jax.experimental import pallas as pl
from jax.experimental.pallas import tpu as pltpu
from jax.experimental.pallas import tpu_sc as plsc

N = 10000
E = 160000
EMBED = 256
H = 256
LEVELS = 3

NSLAB = 4
SLAB = H // NSLAB
PASSES = NSLAB // 2
NC = 2
NS = 16
LANES = 16

ROW_TILES = 10
ROWS_PER_TILE = N // ROW_TILES
EDGES_PER_TILE = E // NS
EB = 80
NBLK = EDGES_PER_TILE // EB

NBLOCK_TC = 400
GRID_TC = N // NBLOCK_TC


def _prep_body(x_ref, w_ref, b_ref, bf_ref, h0_ref, c0_ref, uf_ref,
               xiou_ref, xf2_ref, th_ref, tuc_ref):
    xz = jnp.dot(x_ref[...], w_ref[...], preferred_element_type=jnp.float32)
    xz = xz + b_ref[...]
    xiou_ref[...] = xz[:, : 3 * H]
    xf = -(xz[:, 3 * H:] + bf_ref[...])
    h0 = h0_ref[...]
    hu = jnp.dot(h0, uf_ref[...], preferred_element_type=jnp.float32)
    c0 = c0_ref[...]
    for s in range(NSLAB):
        sl = slice(s * SLAB, (s + 1) * SLAB)
        xf2_ref[s] = xf[:, sl]
        th_ref[s] = h0[:, sl]
        tuc_ref[s, :, :SLAB] = -hu[:, sl]
        tuc_ref[s, :, SLAB:] = c0[:, sl]


def _prep_call(x, W, b2, bf2, h0, c0, U_f):
    return pl.pallas_call(
        _prep_body,
        grid=(GRID_TC,),
        in_specs=[
            pl.BlockSpec((NBLOCK_TC, EMBED), lambda i: (i, 0)),
            pl.BlockSpec((EMBED, 4 * H), lambda i: (0, 0)),
            pl.BlockSpec((1, 4 * H), lambda i: (0, 0)),
            pl.BlockSpec((1, H), lambda i: (0, 0)),
            pl.BlockSpec((NBLOCK_TC, H), lambda i: (i, 0)),
            pl.BlockSpec((NBLOCK_TC, H), lambda i: (i, 0)),
            pl.BlockSpec((H, H), lambda i: (0, 0)),
        ],
        out_specs=[
            pl.BlockSpec((NBLOCK_TC, 3 * H), lambda i: (i, 0)),
            pl.BlockSpec((NSLAB, NBLOCK_TC, SLAB), lambda i: (0, i, 0)),
            pl.BlockSpec((NSLAB, NBLOCK_TC, SLAB), lambda i: (0, i, 0)),
            pl.BlockSpec((NSLAB, NBLOCK_TC, 2 * SLAB), lambda i: (0, i, 0)),
        ],
        out_shape=[
            jax.ShapeDtypeStruct((N, 3 * H), jnp.float32),
            jax.ShapeDtypeStruct((NSLAB, N, SLAB), jnp.float32),
            jax.ShapeDtypeStruct((NSLAB, N, SLAB), jnp.float32),
            jax.ShapeDtypeStruct((NSLAB, N, 2 * SLAB), jnp.float32),
        ],
    )(x, W, b2, bf2, h0, c0, U_f)


def _sc_hsum_body(th_hbm, child_hbm, parent_hbm, zeros_hbm,
                  hsum_hbm,
                  idxc, idxp, thb0, thb1,
                  acc_h, gsem0, gsem1, hsem0, hsem1):
    core = lax.axis_index("c")
    sid = lax.axis_index("s")
    rbase = sid * ROWS_PER_TILE
    thb = (thb0, thb1)
    gsem = (gsem0, gsem1)
    hsem = (hsem0, hsem1)

    pltpu.sync_copy(child_hbm.at[pl.ds(sid * NBLK, NBLK)], idxc)
    pltpu.sync_copy(parent_hbm.at[pl.ds(sid * NBLK, NBLK)], idxp)

    def fire_gather(i, b, slab):
        pltpu.async_copy(th_hbm.at[slab].at[idxc.at[i]], thb[b], gsem[b])

    def wait_gather(b):
        pltpu.make_async_copy(th_hbm.at[0, pl.ds(0, EB)], thb[b],
                              gsem[b]).wait()

    def wait_scatter(i, b):
        pltpu.make_async_copy(thb[b], acc_h.at[idxp.at[i]], hsem[b]).wait()

    for s_local in range(PASSES):
        slab = PASSES * core + s_local

        @pl.when(sid < ROW_TILES)
        def _zero():
            pltpu.sync_copy(zeros_hbm, acc_h.at[pl.ds(rbase, ROWS_PER_TILE)])

        plsc.subcore_barrier()

        fire_gather(0, 0, slab)
        fire_gather(1, 1, slab)

        def blockpair(g2, carry):
            for b in range(2):
                i = 2 * g2 + b
                wait_gather(b)
                pltpu.async_copy(thb[b], acc_h.at[idxp.at[i]], hsem[b],
                                 add=True)
                wait_scatter(i, b)

                @pl.when(i + 2 < NBLK)
                def _fg():
                    fire_gather(i + 2, b, slab)

            return carry

        lax.fori_loop(0, NBLK // 2, blockpair, 0)
        i_last = NBLK - 1
        wait_gather(0)
        pltpu.async_copy(thb[0], acc_h.at[idxp.at[i_last]], hsem[0], add=True)
        wait_scatter(i_last, 0)
        plsc.subcore_barrier()

        @pl.when(sid < ROW_TILES)
        def _copy_out():
            obase = slab * N + rbase
            pltpu.sync_copy(acc_h.at[pl.ds(rbase, ROWS_PER_TILE)],
                            hsum_hbm.at[pl.ds(obase, ROWS_PER_TILE)])

        plsc.subcore_barrier()


_sc_hsum = functools.partial(
    pl.kernel,
    out_type=jax.ShapeDtypeStruct((NSLAB * N, SLAB), jnp.float32),
    mesh=plsc.VectorSubcoreMesh(
        core_axis_name="c", subcore_axis_name="s",
        num_cores=NC, num_subcores=NS),
    compiler_params=pltpu.CompilerParams(use_tc_tiling_on_sc=False),
    scratch_types=[
        pltpu.VMEM((NBLK, EB), jnp.int32),
        pltpu.VMEM((NBLK, EB), jnp.int32),
        pltpu.VMEM((EB, SLAB), jnp.float32),
        pltpu.VMEM((EB, SLAB), jnp.float32),
        pltpu.VMEM_SHARED((N, SLAB), jnp.float32),
        pltpu.SemaphoreType.DMA,
        pltpu.SemaphoreType.DMA,
        pltpu.SemaphoreType.DMA,
        pltpu.SemaphoreType.DMA,
    ],
)(_sc_hsum_body)


def _sc_fec_body(tuc_hbm, xf_hbm, child_hbm, parent_hbm, zeros_hbm,
                 fc_hbm,
                 idxc, idxp, tucb0, tucb1, xfb0, xfb1, fecb0, fecb1,
                 acc_fc, gsem0, gsem1, ssem0, ssem1):
    core = lax.axis_index("c")
    sid = lax.axis_index("s")
    rbase = sid * ROWS_PER_TILE
    tucb = (tucb0, tucb1)
    xfb = (xfb0, xfb1)
    fecb = (fecb0, fecb1)
    gsem = (gsem0, gsem1)
    ssem = (ssem0, ssem1)

    pltpu.sync_copy(child_hbm.at[pl.ds(sid * NBLK, NBLK)], idxc)
    pltpu.sync_copy(parent_hbm.at[pl.ds(sid * NBLK, NBLK)], idxp)

    def fire_gathers(i, b, slab):
        pltpu.async_copy(tuc_hbm.at[slab].at[idxc.at[i]], tucb[b], gsem[b])
        pltpu.async_copy(xf_hbm.at[slab].at[idxp.at[i]], xfb[b], gsem[b])

    def wait_gathers(b):
        pltpu.make_async_copy(tuc_hbm.at[0, pl.ds(0, EB)], tucb[b],
                              gsem[b]).wait()
        pltpu.make_async_copy(xf_hbm.at[0, pl.ds(0, EB)], xfb[b],
                              gsem[b]).wait()

    def wait_fec_scatter(i, b):
        pltpu.make_async_copy(fecb[b], acc_fc.at[idxp.at[i]], ssem[b]).wait()

    def compute(b):
        def row8(r8, carry):
            for rr in range(8):
                r = r8 * 8 + rr
                for g in range(SLAB // LANES):
                    sl = pl.ds(g * LANES, LANES)
                    sl2 = pl.ds(SLAB + g * LANES, LANES)
                    nhu = tucb[b][r, sl]
                    cc = tucb[b][r, sl2]
                    nxf = xfb[b][r, sl]
                    fecb[b][r, sl] = cc / (1.0 + jnp.exp(nxf + nhu))
            return carry

        lax.fori_loop(0, EB // 8, row8, 0)

    for s_local in range(PASSES):
        slab = PASSES * core + s_local

        @pl.when(sid < ROW_TILES)
        def _zero():
            pltpu.sync_copy(zeros_hbm, acc_fc.at[pl.ds(rbase, ROWS_PER_TILE)])

        plsc.subcore_barrier()

        fire_gathers(0, 0, slab)
        fire_gathers(1, 1, slab)

        def blockpair(g2, carry):
            for b in range(2):
                i = 2 * g2 + b
                wait_gathers(b)

                @pl.when(i >= 2)
                def _ws():
                    wait_fec_scatter(i, b)

                compute(b)
                pltpu.async_copy(fecb[b], acc_fc.at[idxp.at[i]], ssem[b],
                                 add=True)

                @pl.when(i + 2 < NBLK)
                def _fg():
                    fire_gathers(i + 2, b, slab)

            return carry

        lax.fori_loop(0, NBLK // 2, blockpair, 0)
        i_last = NBLK - 1
        wait_gathers(0)
        wait_fec_scatter(i_last, 0)
        compute(0)
        pltpu.async_copy(fecb[0], acc_fc.at[idxp.at[i_last]], ssem[0],
                         add=True)
        wait_fec_scatter(i_last, 0)
        wait_fec_scatter(i_last, 1)
        plsc.subcore_barrier()

        @pl.when(sid < ROW_TILES)
        def _copy_out():
            obase = slab * N + rbase
            pltpu.sync_copy(acc_fc.at[pl.ds(rbase, ROWS_PER_TILE)],
                            fc_hbm.at[pl.ds(obase, ROWS_PER_TILE)])

        plsc.subcore_barrier()


_sc_fec = functools.partial(
    pl.kernel,
    out_type=jax.ShapeDtypeStruct((NSLAB * N, SLAB), jnp.float32),
    mesh=plsc.VectorSubcoreMesh(
        core_axis_name="c", subcore_axis_name="s",
        num_cores=NC, num_subcores=NS),
    compiler_params=pltpu.CompilerParams(use_tc_tiling_on_sc=False),
    scratch_types=[
        pltpu.VMEM((NBLK, EB), jnp.int32),
        pltpu.VMEM((NBLK, EB), jnp.int32),
        pltpu.VMEM((EB, 2 * SLAB), jnp.float32),
        pltpu.VMEM((EB, 2 * SLAB), jnp.float32),
        pltpu.VMEM((EB, SLAB), jnp.float32),
        pltpu.VMEM((EB, SLAB), jnp.float32),
        pltpu.VMEM((EB, SLAB), jnp.float32),
        pltpu.VMEM((EB, SLAB), jnp.float32),
        pltpu.VMEM_SHARED((N, SLAB), jnp.float32),
        pltpu.SemaphoreType.DMA,
        pltpu.SemaphoreType.DMA,
        pltpu.SemaphoreType.DMA,
        pltpu.SemaphoreType.DMA,
    ],
)(_sc_fec_body)


def _level_body(hs_ref, fc_ref, xiou_ref, u_ref, uf_ref,
                th_ref, tuc_ref, h_ref, c_ref):
    hs = jnp.concatenate([hs_ref[s] for s in range(NSLAB)], axis=1)
    uz = jnp.dot(hs, u_ref[...], preferred_element_type=jnp.float32)
    xiou = xiou_ref[...]
    i_g = jax.nn.sigmoid(xiou[:, :H] + uz[:, :H])
    o_g = jax.nn.sigmoid(xiou[:, H:2 * H] + uz[:, H:2 * H])
    u_g = jnp.tanh(xiou[:, 2 * H:] + uz[:, 2 * H:])
    fc = jnp.concatenate([fc_ref[s] for s in range(NSLAB)], axis=1)
    c_new = i_g * u_g + fc
    h_new = o_g * jnp.tanh(c_new)
    hu = jnp.dot(h_new, uf_ref[...], preferred_element_type=jnp.float32)
    for s in range(NSLAB):
        sl = slice(s * SLAB, (s + 1) * SLAB)
        th_ref[s] = h_new[:, sl]
        tuc_ref[s, :, :SLAB] = -hu[:, sl]
        tuc_ref[s, :, SLAB:] = c_new[:, sl]
    h_ref[...] = h_new
    c_ref[...] = c_new


def _level_call(hs, fc, xiou, U, U_f):
    return pl.pallas_call(
        _level_body,
        grid=(GRID_TC,),
        in_specs=[
            pl.BlockSpec((NSLAB, NBLOCK_TC, SLAB), lambda i: (0, i, 0)),
            pl.BlockSpec((NSLAB, NBLOCK_TC, SLAB), lambda i: (0, i, 0)),
            pl.BlockSpec((NBLOCK_TC, 3 * H), lambda i: (i, 0)),
            pl.BlockSpec((H, 3 * H), lambda i: (0, 0)),
            pl.BlockSpec((H, H), lambda i: (0, 0)),
        ],
        out_specs=[
            pl.BlockSpec((NSLAB, NBLOCK_TC, SLAB), lambda i: (0, i, 0)),
            pl.BlockSpec((NSLAB, NBLOCK_TC, 2 * SLAB), lambda i: (0, i, 0)),
            pl.BlockSpec((NBLOCK_TC, H), lambda i: (i, 0)),
            pl.BlockSpec((NBLOCK_TC, H), lambda i: (i, 0)),
        ],
        out_shape=[
            jax.ShapeDtypeStruct((NSLAB, N, SLAB), jnp.float32),
            jax.ShapeDtypeStruct((NSLAB, N, 2 * SLAB), jnp.float32),
            jax.ShapeDtypeStruct((N, H), jnp.float32),
            jax.ShapeDtypeStruct((N, H), jnp.float32),
        ],
    )(hs, fc, xiou, U, U_f)


def kernel(x, edge_index, h0, c0, W, U, U_f, b, b_f):
    child = edge_index[0].reshape(E // EB, EB)
    parent = edge_index[1].reshape(E // EB, EB)
    b2 = b.reshape(1, 4 * H)
    bf2 = b_f.reshape(1, H)
    xiou, xf2, th, tuc = _prep_call(x, W, b2, bf2, h0, c0, U_f)
    zeros = jnp.zeros((ROWS_PER_TILE, SLAB), jnp.float32)
    h = c = None
    for _ in range(LEVELS):
        hsum = _sc_hsum(th, child, parent, zeros)
        fcv = _sc_fec(tuc, xf2, child, parent, zeros)
        th, tuc, h, c = _level_call(
            hsum.reshape(NSLAB, N, SLAB),
            fcv.reshape(NSLAB, N, SLAB),
            xiou, U, U_f)
    return h, c


# speedup vs baseline: 1.2454x; 1.2454x over previous
import functools

import jax
import jax.numpy as jnp
from jax import lax
from jax.experimental import pallas as pl
from jax.experimental.pallas import tpu as pltpu
from jax.experimental.pallas import tpu_sc as plsc

N = 10000
E = 160000
EMBED = 256
H = 256
LEVELS = 3

NSLAB = 4
SLAB = H // NSLAB
PSLAB = SLAB // 2
PASSES = NSLAB // 2
NC = 2
NS = 16
LANES = 16

ROW_TILES = 10
ROWS_PER_TILE = N // ROW_TILES
EDGES_PER_TILE = E // NS
EB = 80
NBLK = EDGES_PER_TILE // EB

NBLOCK_TC = 400
GRID_TC = N // NBLOCK_TC

_MASKHI = -65536


def _bf16_top(v):
    u = jax.lax.bitcast_convert_type(v, jnp.int32)
    rounded = u + 0x8000 + jax.lax.shift_right_logical(u, 16) % 2
    return rounded & _MASKHI


def _pack_rows(v):
    lo = jnp.concatenate([v[:, 0:16], v[:, 32:48]], axis=1)
    hi = jnp.concatenate([v[:, 16:32], v[:, 48:64]], axis=1)
    lo_t = jax.lax.shift_right_logical(_bf16_top(lo), 16)
    return _bf16_top(hi) | lo_t


def _prep_body(x_ref, w_ref, b_ref, bf_ref, h0_ref, c0_ref, uf_ref,
               xiou_ref, xf2_ref, th_ref, tuc_ref):
    xz = jnp.dot(x_ref[...], w_ref[...], preferred_element_type=jnp.float32)
    xz = xz + b_ref[...]
    xiou_ref[...] = xz[:, : 3 * H]
    xf = -(xz[:, 3 * H:] + bf_ref[...])
    h0 = h0_ref[...]
    hu = jnp.dot(h0, uf_ref[...], preferred_element_type=jnp.float32)
    c0 = c0_ref[...]
    for s in range(NSLAB):
        sl = slice(s * SLAB, (s + 1) * SLAB)
        xf2_ref[s] = _pack_rows(xf[:, sl])
        th_ref[s] = _pack_rows(h0[:, sl])
        tuc_ref[s, :, :PSLAB] = _pack_rows(-hu[:, sl])
        tuc_ref[s, :, PSLAB:] = _pack_rows(c0[:, sl])


def _prep_call(x, W, b2, bf2, h0, c0, U_f):
    return pl.pallas_call(
        _prep_body,
        grid=(GRID_TC,),
        in_specs=[
            pl.BlockSpec((NBLOCK_TC, EMBED), lambda i: (i, 0)),
            pl.BlockSpec((EMBED, 4 * H), lambda i: (0, 0)),
            pl.BlockSpec((1, 4 * H), lambda i: (0, 0)),
            pl.BlockSpec((1, H), lambda i: (0, 0)),
            pl.BlockSpec((NBLOCK_TC, H), lambda i: (i, 0)),
            pl.BlockSpec((NBLOCK_TC, H), lambda i: (i, 0)),
            pl.BlockSpec((H, H), lambda i: (0, 0)),
        ],
        out_specs=[
            pl.BlockSpec((NBLOCK_TC, 3 * H), lambda i: (i, 0)),
            pl.BlockSpec((NSLAB, NBLOCK_TC, PSLAB), lambda i: (0, i, 0)),
            pl.BlockSpec((NSLAB, NBLOCK_TC, PSLAB), lambda i: (0, i, 0)),
            pl.BlockSpec((NSLAB, NBLOCK_TC, 2 * PSLAB), lambda i: (0, i, 0)),
        ],
        out_shape=[
            jax.ShapeDtypeStruct((N, 3 * H), jnp.float32),
            jax.ShapeDtypeStruct((NSLAB, N, PSLAB), jnp.int32),
            jax.ShapeDtypeStruct((NSLAB, N, PSLAB), jnp.int32),
            jax.ShapeDtypeStruct((NSLAB, N, 2 * PSLAB), jnp.int32),
        ],
    )(x, W, b2, bf2, h0, c0, U_f)


def _unpack_lo(x):
    return plsc.bitcast(jax.lax.shift_left(x, 16), jnp.float32)


def _unpack_hi(x):
    return plsc.bitcast(x & _MASKHI, jnp.float32)


def _sc_hsum_body(th_hbm, child_hbm, parent_hbm, zeros_hbm,
                  hsum_hbm,
                  idxc, idxp, thb0, thb1, thf0, thf1,
                  acc_h, gsem0, gsem1, hsem0, hsem1):
    core = lax.axis_index("c")
    sid = lax.axis_index("s")
    rbase = sid * ROWS_PER_TILE
    thb = (thb0, thb1)
    thf = (thf0, thf1)
    gsem = (gsem0, gsem1)
    hsem = (hsem0, hsem1)

    pltpu.sync_copy(child_hbm.at[pl.ds(sid * NBLK, NBLK)], idxc)
    pltpu.sync_copy(parent_hbm.at[pl.ds(sid * NBLK, NBLK)], idxp)

    def fire_gather(i, b, slab):
        pltpu.async_copy(th_hbm.at[slab].at[idxc.at[i]], thb[b], gsem[b])

    def wait_gather(b):
        pltpu.make_async_copy(th_hbm.at[0, pl.ds(0, EB)], thb[b],
                              gsem[b]).wait()

    def wait_scatter(i, b):
        pltpu.make_async_copy(thf[b], acc_h.at[idxp.at[i]], hsem[b]).wait()

    def unpack(b):
        def row8(r8, carry):
            for rr in range(8):
                r = r8 * 8 + rr
                for g in range(PSLAB // LANES):
                    x = thb[b][r, pl.ds(g * LANES, LANES)]
                    o = 2 * LANES * g
                    thf[b][r, pl.ds(o, LANES)] = _unpack_lo(x)
                    thf[b][r, pl.ds(o + LANES, LANES)] = _unpack_hi(x)
            return carry

        lax.fori_loop(0, EB // 8, row8, 0)

    for s_local in range(PASSES):
        slab = PASSES * core + s_local

        @pl.when(sid < ROW_TILES)
        def _zero():
            pltpu.sync_copy(zeros_hbm, acc_h.at[pl.ds(rbase, ROWS_PER_TILE)])

        plsc.subcore_barrier()

        fire_gather(0, 0, slab)
        fire_gather(1, 1, slab)

        def blockpair(g2, carry):
            for b in range(2):
                i = 2 * g2 + b
                wait_gather(b)

                @pl.when(i >= 2)
                def _ws():
                    wait_scatter(i, b)

                unpack(b)
                pltpu.async_copy(thf[b], acc_h.at[idxp.at[i]], hsem[b],
                                 add=True)

                @pl.when(i + 2 < NBLK)
                def _fg():
                    fire_gather(i + 2, b, slab)

            return carry

        lax.fori_loop(0, NBLK // 2, blockpair, 0)
        i_last = NBLK - 1
        wait_gather(0)
        wait_scatter(i_last, 0)
        unpack(0)
        pltpu.async_copy(thf[0], acc_h.at[idxp.at[i_last]], hsem[0], add=True)
        wait_scatter(i_last, 0)
        wait_scatter(i_last, 1)
        plsc.subcore_barrier()

        @pl.when(sid < ROW_TILES)
        def _copy_out():
            obase = slab * N + rbase
            pltpu.sync_copy(acc_h.at[pl.ds(rbase, ROWS_PER_TILE)],
                            hsum_hbm.at[pl.ds(obase, ROWS_PER_TILE)])

        plsc.subcore_barrier()


_sc_hsum = functools.partial(
    pl.kernel,
    out_type=jax.ShapeDtypeStruct((NSLAB * N, SLAB), jnp.float32),
    mesh=plsc.VectorSubcoreMesh(
        core_axis_name="c", subcore_axis_name="s",
        num_cores=NC, num_subcores=NS),
    compiler_params=pltpu.CompilerParams(use_tc_tiling_on_sc=False, needs_layout_passes=False),
    scratch_types=[
        pltpu.VMEM((NBLK, EB), jnp.int32),
        pltpu.VMEM((NBLK, EB), jnp.int32),
        pltpu.VMEM((EB, PSLAB), jnp.int32),
        pltpu.VMEM((EB, PSLAB), jnp.int32),
        pltpu.VMEM((EB, SLAB), jnp.float32),
        pltpu.VMEM((EB, SLAB), jnp.float32),
        pltpu.VMEM_SHARED((N, SLAB), jnp.float32),
        pltpu.SemaphoreType.DMA,
        pltpu.SemaphoreType.DMA,
        pltpu.SemaphoreType.DMA,
        pltpu.SemaphoreType.DMA,
    ],
)(_sc_hsum_body)


def _sc_fec_body(tuc_hbm, xf_hbm, child_hbm, parent_hbm, zeros_hbm,
                 fc_hbm,
                 idxc, idxp, tucb0, tucb1, xfb0, xfb1, fecb0, fecb1,
                 acc_fc, gsem0, gsem1, ssem0, ssem1):
    core = lax.axis_index("c")
    sid = lax.axis_index("s")
    rbase = sid * ROWS_PER_TILE
    tucb = (tucb0, tucb1)
    xfb = (xfb0, xfb1)
    fecb = (fecb0, fecb1)
    gsem = (gsem0, gsem1)
    ssem = (ssem0, ssem1)

    pltpu.sync_copy(child_hbm.at[pl.ds(sid * NBLK, NBLK)], idxc)
    pltpu.sync_copy(parent_hbm.at[pl.ds(sid * NBLK, NBLK)], idxp)

    def fire_gathers(i, b, slab):
        pltpu.async_copy(tuc_hbm.at[slab].at[idxc.at[i]], tucb[b], gsem[b])
        pltpu.async_copy(xf_hbm.at[slab].at[idxp.at[i]], xfb[b], gsem[b])

    def wait_gathers(b):
        pltpu.make_async_copy(tuc_hbm.at[0, pl.ds(0, EB)], tucb[b],
                              gsem[b]).wait()
        pltpu.make_async_copy(xf_hbm.at[0, pl.ds(0, EB)], xfb[b],
                              gsem[b]).wait()

    def wait_fec_scatter(i, b):
        pltpu.make_async_copy(fecb[b], acc_fc.at[idxp.at[i]], ssem[b]).wait()

    def compute(b):
        def row8(r8, carry):
            for rr in range(8):
                r = r8 * 8 + rr
                for g in range(PSLAB // LANES):
                    sl = pl.ds(g * LANES, LANES)
                    nhu = tucb[b][r, sl]
                    cc = tucb[b][r, pl.ds(PSLAB + g * LANES, LANES)]
                    nxf = xfb[b][r, sl]
                    o = 2 * LANES * g
                    e_lo = jnp.exp(_unpack_lo(nxf) + _unpack_lo(nhu))
                    fecb[b][r, pl.ds(o, LANES)] = (
                        _unpack_lo(cc) / (1.0 + e_lo))
                    e_hi = jnp.exp(_unpack_hi(nxf) + _unpack_hi(nhu))
                    fecb[b][r, pl.ds(o + LANES, LANES)] = (
                        _unpack_hi(cc) / (1.0 + e_hi))
            return carry

        lax.fori_loop(0, EB // 8, row8, 0)

    for s_local in range(PASSES):
        slab = PASSES * core + s_local

        @pl.when(sid < ROW_TILES)
        def _zero():
            pltpu.sync_copy(zeros_hbm, acc_fc.at[pl.ds(rbase, ROWS_PER_TILE)])

        plsc.subcore_barrier()

        fire_gathers(0, 0, slab)
        fire_gathers(1, 1, slab)

        def blockpair(g2, carry):
            for b in range(2):
                i = 2 * g2 + b
                wait_gathers(b)

                @pl.when(i >= 2)
                def _ws():
                    wait_fec_scatter(i, b)

                compute(b)
                pltpu.async_copy(fecb[b], acc_fc.at[idxp.at[i]], ssem[b],
                                 add=True)

                @pl.when(i + 2 < NBLK)
                def _fg():
                    fire_gathers(i + 2, b, slab)

            return carry

        lax.fori_loop(0, NBLK // 2, blockpair, 0)
        i_last = NBLK - 1
        wait_gathers(0)
        wait_fec_scatter(i_last, 0)
        compute(0)
        pltpu.async_copy(fecb[0], acc_fc.at[idxp.at[i_last]], ssem[0],
                         add=True)
        wait_fec_scatter(i_last, 0)
        wait_fec_scatter(i_last, 1)
        plsc.subcore_barrier()

        @pl.when(sid < ROW_TILES)
        def _copy_out():
            obase = slab * N + rbase
            pltpu.sync_copy(acc_fc.at[pl.ds(rbase, ROWS_PER_TILE)],
                            fc_hbm.at[pl.ds(obase, ROWS_PER_TILE)])

        plsc.subcore_barrier()


_sc_fec = functools.partial(
    pl.kernel,
    out_type=jax.ShapeDtypeStruct((NSLAB * N, SLAB), jnp.float32),
    mesh=plsc.VectorSubcoreMesh(
        core_axis_name="c", subcore_axis_name="s",
        num_cores=NC, num_subcores=NS),
    compiler_params=pltpu.CompilerParams(use_tc_tiling_on_sc=False, needs_layout_passes=False),
    scratch_types=[
        pltpu.VMEM((NBLK, EB), jnp.int32),
        pltpu.VMEM((NBLK, EB), jnp.int32),
        pltpu.VMEM((EB, 2 * PSLAB), jnp.int32),
        pltpu.VMEM((EB, 2 * PSLAB), jnp.int32),
        pltpu.VMEM((EB, PSLAB), jnp.int32),
        pltpu.VMEM((EB, PSLAB), jnp.int32),
        pltpu.VMEM((EB, SLAB), jnp.float32),
        pltpu.VMEM((EB, SLAB), jnp.float32),
        pltpu.VMEM_SHARED((N, SLAB), jnp.float32),
        pltpu.SemaphoreType.DMA,
        pltpu.SemaphoreType.DMA,
        pltpu.SemaphoreType.DMA,
        pltpu.SemaphoreType.DMA,
    ],
)(_sc_fec_body)


def _level_body(hs_ref, fc_ref, xiou_ref, u_ref, uf_ref,
                th_ref, tuc_ref, h_ref, c_ref):
    hs = jnp.concatenate([hs_ref[s] for s in range(NSLAB)], axis=1)
    uz = jnp.dot(hs, u_ref[...], preferred_element_type=jnp.float32)
    xiou = xiou_ref[...]
    i_g = jax.nn.sigmoid(xiou[:, :H] + uz[:, :H])
    o_g = jax.nn.sigmoid(xiou[:, H:2 * H] + uz[:, H:2 * H])
    u_g = jnp.tanh(xiou[:, 2 * H:] + uz[:, 2 * H:])
    fc = jnp.concatenate([fc_ref[s] for s in range(NSLAB)], axis=1)
    c_new = i_g * u_g + fc
    h_new = o_g * jnp.tanh(c_new)
    hu = jnp.dot(h_new, uf_ref[...], preferred_element_type=jnp.float32)
    for s in range(NSLAB):
        sl = slice(s * SLAB, (s + 1) * SLAB)
        th_ref[s] = _pack_rows(h_new[:, sl])
        tuc_ref[s, :, :PSLAB] = _pack_rows(-hu[:, sl])
        tuc_ref[s, :, PSLAB:] = _pack_rows(c_new[:, sl])
    h_ref[...] = h_new
    c_ref[...] = c_new


def _level_call(hs, fc, xiou, U, U_f):
    return pl.pallas_call(
        _level_body,
        grid=(GRID_TC,),
        in_specs=[
            pl.BlockSpec((NSLAB, NBLOCK_TC, SLAB), lambda i: (0, i, 0)),
            pl.BlockSpec((NSLAB, NBLOCK_TC, SLAB), lambda i: (0, i, 0)),
            pl.BlockSpec((NBLOCK_TC, 3 * H), lambda i: (i, 0)),
            pl.BlockSpec((H, 3 * H), lambda i: (0, 0)),
            pl.BlockSpec((H, H), lambda i: (0, 0)),
        ],
        out_specs=[
            pl.BlockSpec((NSLAB, NBLOCK_TC, PSLAB), lambda i: (0, i, 0)),
            pl.BlockSpec((NSLAB, NBLOCK_TC, 2 * PSLAB), lambda i: (0, i, 0)),
            pl.BlockSpec((NBLOCK_TC, H), lambda i: (i, 0)),
            pl.BlockSpec((NBLOCK_TC, H), lambda i: (i, 0)),
        ],
        out_shape=[
            jax.ShapeDtypeStruct((NSLAB, N, PSLAB), jnp.int32),
            jax.ShapeDtypeStruct((NSLAB, N, 2 * PSLAB), jnp.int32),
            jax.ShapeDtypeStruct((N, H), jnp.float32),
            jax.ShapeDtypeStruct((N, H), jnp.float32),
        ],
    )(hs, fc, xiou, U, U_f)


def kernel(x, edge_index, h0, c0, W, U, U_f, b, b_f):
    child = edge_index[0].reshape(E // EB, EB)
    parent = edge_index[1].reshape(E // EB, EB)
    b2 = b.reshape(1, 4 * H)
    bf2 = b_f.reshape(1, H)
    xiou, xf2, th, tuc = _prep_call(x, W, b2, bf2, h0, c0, U_f)
    zeros = jnp.zeros((ROWS_PER_TILE, SLAB), jnp.float32)
    h = c = None
    for _ in range(LEVELS):
        hsum = _sc_hsum(th, child, parent, zeros)
        fcv = _sc_fec(tuc, xf2, child, parent, zeros)
        th, tuc, h, c = _level_call(
            hsum.reshape(NSLAB, N, SLAB),
            fcv.reshape(NSLAB, N, SLAB),
            xiou, U, U_f)
    return h, c


# speedup vs baseline: 1.2571x; 1.0095x over previous
import functools

import jax
import jax.numpy as jnp
from jax import lax
from jax.experimental import pallas as pl
from jax.experimental.pallas import tpu as pltpu
from jax.experimental.pallas import tpu_sc as plsc

N = 10000
E = 160000
EMBED = 256
H = 256
LEVELS = 3

NSLAB = 4
SLAB = H // NSLAB
PSLAB = SLAB // 2
PASSES = NSLAB // 2
NC = 2
NS = 16
LANES = 16

ROW_TILES = 10
ROWS_PER_TILE = N // ROW_TILES
EDGES_PER_TILE = E // NS
EB = 80
NBLK = EDGES_PER_TILE // EB

NBLOCK_TC = 400
GRID_TC = N // NBLOCK_TC

_MASKHI = -65536


def _bf16_top(v):
    u = jax.lax.bitcast_convert_type(v, jnp.int32)
    rounded = u + 0x8000 + jax.lax.shift_right_logical(u, 16) % 2
    return rounded & _MASKHI


def _pack_rows(v):
    lo = jnp.concatenate([v[:, 0:16], v[:, 32:48]], axis=1)
    hi = jnp.concatenate([v[:, 16:32], v[:, 48:64]], axis=1)
    lo_t = jax.lax.shift_right_logical(_bf16_top(lo), 16)
    return _bf16_top(hi) | lo_t


def _prep_body(x_ref, w_ref, b_ref, bf_ref, h0_ref, c0_ref, uf_ref,
               xiou_ref, xf2_ref, th_ref, tuc_ref):
    xz = jnp.dot(x_ref[...], w_ref[...], preferred_element_type=jnp.float32)
    xz = xz + b_ref[...]
    xiou_ref[...] = xz[:, : 3 * H]
    xf = -(xz[:, 3 * H:] + bf_ref[...])
    h0 = h0_ref[...]
    hu = jnp.dot(h0, uf_ref[...], preferred_element_type=jnp.float32)
    c0 = c0_ref[...]
    for s in range(NSLAB):
        sl = slice(s * SLAB, (s + 1) * SLAB)
        xf2_ref[s] = _pack_rows(xf[:, sl])
        th_ref[s] = _pack_rows(h0[:, sl])
        tuc_ref[s, :, :PSLAB] = _pack_rows(-hu[:, sl])
        tuc_ref[s, :, PSLAB:] = jax.lax.bitcast_convert_type(
            c0[:, sl], jnp.int32)


def _prep_call(x, W, b2, bf2, h0, c0, U_f):
    return pl.pallas_call(
        _prep_body,
        grid=(GRID_TC,),
        in_specs=[
            pl.BlockSpec((NBLOCK_TC, EMBED), lambda i: (i, 0)),
            pl.BlockSpec((EMBED, 4 * H), lambda i: (0, 0)),
            pl.BlockSpec((1, 4 * H), lambda i: (0, 0)),
            pl.BlockSpec((1, H), lambda i: (0, 0)),
            pl.BlockSpec((NBLOCK_TC, H), lambda i: (i, 0)),
            pl.BlockSpec((NBLOCK_TC, H), lambda i: (i, 0)),
            pl.BlockSpec((H, H), lambda i: (0, 0)),
        ],
        out_specs=[
            pl.BlockSpec((NBLOCK_TC, 3 * H), lambda i: (i, 0)),
            pl.BlockSpec((NSLAB, NBLOCK_TC, PSLAB), lambda i: (0, i, 0)),
            pl.BlockSpec((NSLAB, NBLOCK_TC, PSLAB), lambda i: (0, i, 0)),
            pl.BlockSpec((NSLAB, NBLOCK_TC, PSLAB + SLAB), lambda i: (0, i, 0)),
        ],
        out_shape=[
            jax.ShapeDtypeStruct((N, 3 * H), jnp.float32),
            jax.ShapeDtypeStruct((NSLAB, N, PSLAB), jnp.int32),
            jax.ShapeDtypeStruct((NSLAB, N, PSLAB), jnp.int32),
            jax.ShapeDtypeStruct((NSLAB, N, PSLAB + SLAB), jnp.int32),
        ],
    )(x, W, b2, bf2, h0, c0, U_f)


def _unpack_lo(x):
    return plsc.bitcast(jax.lax.shift_left(x, 16), jnp.float32)


def _unpack_hi(x):
    return plsc.bitcast(x & _MASKHI, jnp.float32)


def _sc_hsum_body(th_hbm, child_hbm, parent_hbm, zeros_hbm,
                  hsum_hbm,
                  idxc, idxp, thb0, thb1, thf0, thf1,
                  acc_h, gsem0, gsem1, hsem0, hsem1):
    core = lax.axis_index("c")
    sid = lax.axis_index("s")
    rbase = sid * ROWS_PER_TILE
    thb = (thb0, thb1)
    thf = (thf0, thf1)
    gsem = (gsem0, gsem1)
    hsem = (hsem0, hsem1)

    pltpu.sync_copy(child_hbm.at[pl.ds(sid * NBLK, NBLK)], idxc)
    pltpu.sync_copy(parent_hbm.at[pl.ds(sid * NBLK, NBLK)], idxp)

    def fire_gather(i, b, slab):
        pltpu.async_copy(th_hbm.at[slab].at[idxc.at[i]], thb[b], gsem[b])

    def wait_gather(b):
        pltpu.make_async_copy(th_hbm.at[0, pl.ds(0, EB)], thb[b],
                              gsem[b]).wait()

    def wait_scatter(i, b):
        pltpu.make_async_copy(thf[b], acc_h.at[idxp.at[i]], hsem[b]).wait()

    def unpack(b):
        def row8(r8, carry):
            for rr in range(8):
                r = r8 * 8 + rr
                for g in range(PSLAB // LANES):
                    x = thb[b][r, pl.ds(g * LANES, LANES)]
                    o = 2 * LANES * g
                    thf[b][r, pl.ds(o, LANES)] = _unpack_lo(x)
                    thf[b][r, pl.ds(o + LANES, LANES)] = _unpack_hi(x)
            return carry

        lax.fori_loop(0, EB // 8, row8, 0)

    for s_local in range(PASSES):
        slab = PASSES * core + s_local

        @pl.when(sid < ROW_TILES)
        def _zero():
            pltpu.sync_copy(zeros_hbm, acc_h.at[pl.ds(rbase, ROWS_PER_TILE)])

        plsc.subcore_barrier()

        fire_gather(0, 0, slab)
        fire_gather(1, 1, slab)

        def blockpair(g2, carry):
            for b in range(2):
                i = 2 * g2 + b
                wait_gather(b)

                @pl.when(i >= 2)
                def _ws():
                    wait_scatter(i, b)

                unpack(b)
                pltpu.async_copy(thf[b], acc_h.at[idxp.at[i]], hsem[b],
                                 add=True)

                @pl.when(i + 2 < NBLK)
                def _fg():
                    fire_gather(i + 2, b, slab)

            return carry

        lax.fori_loop(0, NBLK // 2, blockpair, 0)
        i_last = NBLK - 1
        wait_gather(0)
        wait_scatter(i_last, 0)
        unpack(0)
        pltpu.async_copy(thf[0], acc_h.at[idxp.at[i_last]], hsem[0], add=True)
        wait_scatter(i_last, 0)
        wait_scatter(i_last, 1)
        plsc.subcore_barrier()

        @pl.when(sid < ROW_TILES)
        def _copy_out():
            obase = slab * N + rbase
            pltpu.sync_copy(acc_h.at[pl.ds(rbase, ROWS_PER_TILE)],
                            hsum_hbm.at[pl.ds(obase, ROWS_PER_TILE)])

        plsc.subcore_barrier()


_sc_hsum = functools.partial(
    pl.kernel,
    out_type=jax.ShapeDtypeStruct((NSLAB * N, SLAB), jnp.float32),
    mesh=plsc.VectorSubcoreMesh(
        core_axis_name="c", subcore_axis_name="s",
        num_cores=NC, num_subcores=NS),
    compiler_params=pltpu.CompilerParams(use_tc_tiling_on_sc=False, needs_layout_passes=False),
    scratch_types=[
        pltpu.VMEM((NBLK, EB), jnp.int32),
        pltpu.VMEM((NBLK, EB), jnp.int32),
        pltpu.VMEM((EB, PSLAB), jnp.int32),
        pltpu.VMEM((EB, PSLAB), jnp.int32),
        pltpu.VMEM((EB, SLAB), jnp.float32),
        pltpu.VMEM((EB, SLAB), jnp.float32),
        pltpu.VMEM_SHARED((N, SLAB), jnp.float32),
        pltpu.SemaphoreType.DMA,
        pltpu.SemaphoreType.DMA,
        pltpu.SemaphoreType.DMA,
        pltpu.SemaphoreType.DMA,
    ],
)(_sc_hsum_body)


def _sc_fec_body(tuc_hbm, xf_hbm, child_hbm, parent_hbm, zeros_hbm,
                 fc_hbm,
                 idxc, idxp, tucb0, tucb1, xfb0, xfb1, fecb0, fecb1,
                 acc_fc, gsem0, gsem1, ssem0, ssem1):
    core = lax.axis_index("c")
    sid = lax.axis_index("s")
    rbase = sid * ROWS_PER_TILE
    tucb = (tucb0, tucb1)
    xfb = (xfb0, xfb1)
    fecb = (fecb0, fecb1)
    gsem = (gsem0, gsem1)
    ssem = (ssem0, ssem1)

    pltpu.sync_copy(child_hbm.at[pl.ds(sid * NBLK, NBLK)], idxc)
    pltpu.sync_copy(parent_hbm.at[pl.ds(sid * NBLK, NBLK)], idxp)

    def fire_gathers(i, b, slab):
        pltpu.async_copy(tuc_hbm.at[slab].at[idxc.at[i]], tucb[b], gsem[b])
        pltpu.async_copy(xf_hbm.at[slab].at[idxp.at[i]], xfb[b], gsem[b])

    def wait_gathers(b):
        pltpu.make_async_copy(tuc_hbm.at[0, pl.ds(0, EB)], tucb[b],
                              gsem[b]).wait()
        pltpu.make_async_copy(xf_hbm.at[0, pl.ds(0, EB)], xfb[b],
                              gsem[b]).wait()

    def wait_fec_scatter(i, b):
        pltpu.make_async_copy(fecb[b], acc_fc.at[idxp.at[i]], ssem[b]).wait()

    def compute(b):
        def row8(r8, carry):
            for rr in range(8):
                r = r8 * 8 + rr
                for g in range(PSLAB // LANES):
                    sl = pl.ds(g * LANES, LANES)
                    nhu = tucb[b][r, sl]
                    nxf = xfb[b][r, sl]
                    o = 2 * LANES * g
                    cc_lo = plsc.bitcast(
                        tucb[b][r, pl.ds(PSLAB + o, LANES)], jnp.float32)
                    cc_hi = plsc.bitcast(
                        tucb[b][r, pl.ds(PSLAB + o + LANES, LANES)],
                        jnp.float32)
                    e_lo = jnp.exp(_unpack_lo(nxf) + _unpack_lo(nhu))
                    fecb[b][r, pl.ds(o, LANES)] = cc_lo / (1.0 + e_lo)
                    e_hi = jnp.exp(_unpack_hi(nxf) + _unpack_hi(nhu))
                    fecb[b][r, pl.ds(o + LANES, LANES)] = cc_hi / (1.0 + e_hi)
            return carry

        lax.fori_loop(0, EB // 8, row8, 0)

    for s_local in range(PASSES):
        slab = PASSES * core + s_local

        @pl.when(sid < ROW_TILES)
        def _zero():
            pltpu.sync_copy(zeros_hbm, acc_fc.at[pl.ds(rbase, ROWS_PER_TILE)])

        plsc.subcore_barrier()

        fire_gathers(0, 0, slab)
        fire_gathers(1, 1, slab)

        def blockpair(g2, carry):
            for b in range(2):
                i = 2 * g2 + b
                wait_gathers(b)

                @pl.when(i >= 2)
                def _ws():
                    wait_fec_scatter(i, b)

                compute(b)
                pltpu.async_copy(fecb[b], acc_fc.at[idxp.at[i]], ssem[b],
                                 add=True)

                @pl.when(i + 2 < NBLK)
                def _fg():
                    fire_gathers(i + 2, b, slab)

            return carry

        lax.fori_loop(0, NBLK // 2, blockpair, 0)
        i_last = NBLK - 1
        wait_gathers(0)
        wait_fec_scatter(i_last, 0)
        compute(0)
        pltpu.async_copy(fecb[0], acc_fc.at[idxp.at[i_last]], ssem[0],
                         add=True)
        wait_fec_scatter(i_last, 0)
        wait_fec_scatter(i_last, 1)
        plsc.subcore_barrier()

        @pl.when(sid < ROW_TILES)
        def _copy_out():
            obase = slab * N + rbase
            pltpu.sync_copy(acc_fc.at[pl.ds(rbase, ROWS_PER_TILE)],
                            fc_hbm.at[pl.ds(obase, ROWS_PER_TILE)])

        plsc.subcore_barrier()


_sc_fec = functools.partial(
    pl.kernel,
    out_type=jax.ShapeDtypeStruct((NSLAB * N, SLAB), jnp.float32),
    mesh=plsc.VectorSubcoreMesh(
        core_axis_name="c", subcore_axis_name="s",
        num_cores=NC, num_subcores=NS),
    compiler_params=pltpu.CompilerParams(use_tc_tiling_on_sc=False, needs_layout_passes=False),
    scratch_types=[
        pltpu.VMEM((NBLK, EB), jnp.int32),
        pltpu.VMEM((NBLK, EB), jnp.int32),
        pltpu.VMEM((EB, PSLAB + SLAB), jnp.int32),
        pltpu.VMEM((EB, PSLAB + SLAB), jnp.int32),
        pltpu.VMEM((EB, PSLAB), jnp.int32),
        pltpu.VMEM((EB, PSLAB), jnp.int32),
        pltpu.VMEM((EB, SLAB), jnp.float32),
        pltpu.VMEM((EB, SLAB), jnp.float32),
        pltpu.VMEM_SHARED((N, SLAB), jnp.float32),
        pltpu.SemaphoreType.DMA,
        pltpu.SemaphoreType.DMA,
        pltpu.SemaphoreType.DMA,
        pltpu.SemaphoreType.DMA,
    ],
)(_sc_fec_body)


def _level_body(hs_ref, fc_ref, xiou_ref, u_ref, uf_ref,
                th_ref, tuc_ref, h_ref, c_ref):
    hs = jnp.concatenate([hs_ref[s] for s in range(NSLAB)], axis=1)
    uz = jnp.dot(hs, u_ref[...], preferred_element_type=jnp.float32)
    xiou = xiou_ref[...]
    i_g = jax.nn.sigmoid(xiou[:, :H] + uz[:, :H])
    o_g = jax.nn.sigmoid(xiou[:, H:2 * H] + uz[:, H:2 * H])
    u_g = jnp.tanh(xiou[:, 2 * H:] + uz[:, 2 * H:])
    fc = jnp.concatenate([fc_ref[s] for s in range(NSLAB)], axis=1)
    c_new = i_g * u_g + fc
    h_new = o_g * jnp.tanh(c_new)
    hu = jnp.dot(h_new, uf_ref[...], preferred_element_type=jnp.float32)
    for s in range(NSLAB):
        sl = slice(s * SLAB, (s + 1) * SLAB)
        th_ref[s] = _pack_rows(h_new[:, sl])
        tuc_ref[s, :, :PSLAB] = _pack_rows(-hu[:, sl])
        tuc_ref[s, :, PSLAB:] = jax.lax.bitcast_convert_type(
            c_new[:, sl], jnp.int32)
    h_ref[...] = h_new
    c_ref[...] = c_new


def _level_call(hs, fc, xiou, U, U_f):
    return pl.pallas_call(
        _level_body,
        grid=(GRID_TC,),
        in_specs=[
            pl.BlockSpec((NSLAB, NBLOCK_TC, SLAB), lambda i: (0, i, 0)),
            pl.BlockSpec((NSLAB, NBLOCK_TC, SLAB), lambda i: (0, i, 0)),
            pl.BlockSpec((NBLOCK_TC, 3 * H), lambda i: (i, 0)),
            pl.BlockSpec((H, 3 * H), lambda i: (0, 0)),
            pl.BlockSpec((H, H), lambda i: (0, 0)),
        ],
        out_specs=[
            pl.BlockSpec((NSLAB, NBLOCK_TC, PSLAB), lambda i: (0, i, 0)),
            pl.BlockSpec((NSLAB, NBLOCK_TC, PSLAB + SLAB), lambda i: (0, i, 0)),
            pl.BlockSpec((NBLOCK_TC, H), lambda i: (i, 0)),
            pl.BlockSpec((NBLOCK_TC, H), lambda i: (i, 0)),
        ],
        out_shape=[
            jax.ShapeDtypeStruct((NSLAB, N, PSLAB), jnp.int32),
            jax.ShapeDtypeStruct((NSLAB, N, PSLAB + SLAB), jnp.int32),
            jax.ShapeDtypeStruct((N, H), jnp.float32),
            jax.ShapeDtypeStruct((N, H), jnp.float32),
        ],
    )(hs, fc, xiou, U, U_f)


def kernel(x, edge_index, h0, c0, W, U, U_f, b, b_f):
    child = edge_index[0].reshape(E // EB, EB)
    parent = edge_index[1].reshape(E // EB, EB)
    b2 = b.reshape(1, 4 * H)
    bf2 = b_f.reshape(1, H)
    xiou, xf2, th, tuc = _prep_call(x, W, b2, bf2, h0, c0, U_f)
    zeros = jnp.zeros((ROWS_PER_TILE, SLAB), jnp.float32)
    h = c = None
    for _ in range(LEVELS):
        hsum = _sc_hsum(th, child, parent, zeros)
        fcv = _sc_fec(tuc, xf2, child, parent, zeros)
        th, tuc, h, c = _level_call(
            hsum.reshape(NSLAB, N, SLAB),
            fcv.reshape(NSLAB, N, SLAB),
            xiou, U, U_f)
    return h, c


# speedup vs baseline: 3.6070x; 2.8692x over previous
import functools

import jax
import jax.numpy as jnp
from jax import lax
from jax.experimental import pallas as pl
from jax.experimental.pallas import tpu as pltpu
from jax.experimental.pallas import tpu_sc as plsc

N = 10000
E = 160000
EMBED = 256
H = 256
LEVELS = 3

NSLAB = 4
SLAB = H // NSLAB
PSLAB = SLAB // 2
PASSES = NSLAB // 2
NC = 2
NS = 16
LANES = 16

ROW_TILES = 10
ROWS_PER_TILE = N // ROW_TILES
EDGES_PER_TILE = E // NS
EB = 80
NBLK = EDGES_PER_TILE // EB

NBLOCK_TC = 400
GRID_TC = N // NBLOCK_TC

_MASKHI = -65536


def _bf16_top(v):
    u = jax.lax.bitcast_convert_type(v, jnp.int32)
    rounded = u + 0x8000 + jax.lax.shift_right_logical(u, 16) % 2
    return rounded & _MASKHI


def _pack_rows(v):
    lo = jnp.concatenate([v[:, 0:16], v[:, 32:48]], axis=1)
    hi = jnp.concatenate([v[:, 16:32], v[:, 48:64]], axis=1)
    lo_t = jax.lax.shift_right_logical(_bf16_top(lo), 16)
    return _bf16_top(hi) | lo_t


def _prep_body(x_ref, w_ref, b_ref, bf_ref, h0_ref, c0_ref, uf_ref,
               xiou_ref, xf2_ref, th_ref, tuc_ref):
    xz = jnp.dot(x_ref[...], w_ref[...], preferred_element_type=jnp.float32)
    xz = xz + b_ref[...]
    xiou_ref[...] = xz[:, : 3 * H]
    xf = -(xz[:, 3 * H:] + bf_ref[...])
    h0 = h0_ref[...]
    hu = jnp.dot(h0, uf_ref[...], preferred_element_type=jnp.float32)
    c0 = c0_ref[...]
    for s in range(NSLAB):
        sl = slice(s * SLAB, (s + 1) * SLAB)
        xf2_ref[s] = _pack_rows(xf[:, sl])
        th_ref[s] = _pack_rows(h0[:, sl])
        tuc_ref[s, :, :PSLAB] = _pack_rows(-hu[:, sl])
        tuc_ref[s, :, PSLAB:] = jax.lax.bitcast_convert_type(
            c0[:, sl], jnp.int32)


def _prep_call(x, W, b2, bf2, h0, c0, U_f):
    return pl.pallas_call(
        _prep_body,
        grid=(GRID_TC,),
        in_specs=[
            pl.BlockSpec((NBLOCK_TC, EMBED), lambda i: (i, 0)),
            pl.BlockSpec((EMBED, 4 * H), lambda i: (0, 0)),
            pl.BlockSpec((1, 4 * H), lambda i: (0, 0)),
            pl.BlockSpec((1, H), lambda i: (0, 0)),
            pl.BlockSpec((NBLOCK_TC, H), lambda i: (i, 0)),
            pl.BlockSpec((NBLOCK_TC, H), lambda i: (i, 0)),
            pl.BlockSpec((H, H), lambda i: (0, 0)),
        ],
        out_specs=[
            pl.BlockSpec((NBLOCK_TC, 3 * H), lambda i: (i, 0)),
            pl.BlockSpec((NSLAB, NBLOCK_TC, PSLAB), lambda i: (0, i, 0)),
            pl.BlockSpec((NSLAB, NBLOCK_TC, PSLAB), lambda i: (0, i, 0)),
            pl.BlockSpec((NSLAB, NBLOCK_TC, PSLAB + SLAB), lambda i: (0, i, 0)),
        ],
        out_shape=[
            jax.ShapeDtypeStruct((N, 3 * H), jnp.float32),
            jax.ShapeDtypeStruct((NSLAB, N, PSLAB), jnp.int32),
            jax.ShapeDtypeStruct((NSLAB, N, PSLAB), jnp.int32),
            jax.ShapeDtypeStruct((NSLAB, N, PSLAB + SLAB), jnp.int32),
        ],
    )(x, W, b2, bf2, h0, c0, U_f)


def _unpack_lo(x):
    return plsc.bitcast(jax.lax.shift_left(x, 16), jnp.float32)


def _unpack_hi(x):
    return plsc.bitcast(x & _MASKHI, jnp.float32)


def _sc_hsum_body(th_hbm, child_hbm, parent_hbm, zeros_hbm,
                  hsum_hbm,
                  idxc, idxp, thb0, thb1, thf0, thf1,
                  acc_h, gsem0, gsem1, hsem0, hsem1):
    core = lax.axis_index("c")
    sid = lax.axis_index("s")
    rbase = sid * ROWS_PER_TILE
    thb = (thb0, thb1)
    thf = (thf0, thf1)
    gsem = (gsem0, gsem1)
    hsem = (hsem0, hsem1)

    pltpu.sync_copy(child_hbm.at[pl.ds(sid * NBLK, NBLK)], idxc)
    pltpu.sync_copy(parent_hbm.at[pl.ds(sid * NBLK, NBLK)], idxp)

    def fire_gather(i, b, slab):
        pltpu.async_copy(th_hbm.at[slab].at[idxc.at[i]], thb[b], gsem[b])

    def wait_gather(b):
        pltpu.make_async_copy(th_hbm.at[0, pl.ds(0, EB)], thb[b],
                              gsem[b]).wait()

    def wait_scatter(i, b):
        pltpu.make_async_copy(thf[b], acc_h.at[idxp.at[i]], hsem[b]).wait()

    def unpack(b):
        @plsc.parallel_loop(0, EB, step=1, unroll=8)
        def _rows(r):
            for g in range(PSLAB // LANES):
                x = thb[b][r, pl.ds(g * LANES, LANES)]
                o = 2 * LANES * g
                thf[b][r, pl.ds(o, LANES)] = _unpack_lo(x)
                thf[b][r, pl.ds(o + LANES, LANES)] = _unpack_hi(x)

    for s_local in range(PASSES):
        slab = PASSES * core + s_local

        @pl.when(sid < ROW_TILES)
        def _zero():
            pltpu.sync_copy(zeros_hbm, acc_h.at[pl.ds(rbase, ROWS_PER_TILE)])

        plsc.subcore_barrier()

        fire_gather(0, 0, slab)
        fire_gather(1, 1, slab)

        def blockpair(g2, carry):
            for b in range(2):
                i = 2 * g2 + b
                wait_gather(b)

                @pl.when(i >= 2)
                def _ws():
                    wait_scatter(i, b)

                unpack(b)
                pltpu.async_copy(thf[b], acc_h.at[idxp.at[i]], hsem[b],
                                 add=True)

                @pl.when(i + 2 < NBLK)
                def _fg():
                    fire_gather(i + 2, b, slab)

            return carry

        lax.fori_loop(0, NBLK // 2, blockpair, 0)
        i_last = NBLK - 1
        wait_gather(0)
        wait_scatter(i_last, 0)
        unpack(0)
        pltpu.async_copy(thf[0], acc_h.at[idxp.at[i_last]], hsem[0], add=True)
        wait_scatter(i_last, 0)
        wait_scatter(i_last, 1)
        plsc.subcore_barrier()

        @pl.when(sid < ROW_TILES)
        def _copy_out():
            obase = slab * N + rbase
            pltpu.sync_copy(acc_h.at[pl.ds(rbase, ROWS_PER_TILE)],
                            hsum_hbm.at[pl.ds(obase, ROWS_PER_TILE)])

        plsc.subcore_barrier()


_sc_hsum = functools.partial(
    pl.kernel,
    out_type=jax.ShapeDtypeStruct((NSLAB * N, SLAB), jnp.float32),
    mesh=plsc.VectorSubcoreMesh(
        core_axis_name="c", subcore_axis_name="s",
        num_cores=NC, num_subcores=NS),
    compiler_params=pltpu.CompilerParams(use_tc_tiling_on_sc=False, needs_layout_passes=False),
    scratch_types=[
        pltpu.VMEM((NBLK, EB), jnp.int32),
        pltpu.VMEM((NBLK, EB), jnp.int32),
        pltpu.VMEM((EB, PSLAB), jnp.int32),
        pltpu.VMEM((EB, PSLAB), jnp.int32),
        pltpu.VMEM((EB, SLAB), jnp.float32),
        pltpu.VMEM((EB, SLAB), jnp.float32),
        pltpu.VMEM_SHARED((N, SLAB), jnp.float32),
        pltpu.SemaphoreType.DMA,
        pltpu.SemaphoreType.DMA,
        pltpu.SemaphoreType.DMA,
        pltpu.SemaphoreType.DMA,
    ],
)(_sc_hsum_body)


def _sc_fec_body(tuc_hbm, xf_hbm, child_hbm, parent_hbm, zeros_hbm,
                 fc_hbm,
                 idxc, idxp, tucb0, tucb1, xfb0, xfb1, fecb0, fecb1,
                 acc_fc, gsem0, gsem1, ssem0, ssem1):
    core = lax.axis_index("c")
    sid = lax.axis_index("s")
    rbase = sid * ROWS_PER_TILE
    tucb = (tucb0, tucb1)
    xfb = (xfb0, xfb1)
    fecb = (fecb0, fecb1)
    gsem = (gsem0, gsem1)
    ssem = (ssem0, ssem1)

    pltpu.sync_copy(child_hbm.at[pl.ds(sid * NBLK, NBLK)], idxc)
    pltpu.sync_copy(parent_hbm.at[pl.ds(sid * NBLK, NBLK)], idxp)

    def fire_gathers(i, b, slab):
        pltpu.async_copy(tuc_hbm.at[slab].at[idxc.at[i]], tucb[b], gsem[b])
        pltpu.async_copy(xf_hbm.at[slab].at[idxp.at[i]], xfb[b], gsem[b])

    def wait_gathers(b):
        pltpu.make_async_copy(tuc_hbm.at[0, pl.ds(0, EB)], tucb[b],
                              gsem[b]).wait()
        pltpu.make_async_copy(xf_hbm.at[0, pl.ds(0, EB)], xfb[b],
                              gsem[b]).wait()

    def wait_fec_scatter(i, b):
        pltpu.make_async_copy(fecb[b], acc_fc.at[idxp.at[i]], ssem[b]).wait()

    def compute(b):
        @plsc.parallel_loop(0, EB, step=1, unroll=8)
        def _rows(r):
            for g in range(PSLAB // LANES):
                sl = pl.ds(g * LANES, LANES)
                nhu = tucb[b][r, sl]
                nxf = xfb[b][r, sl]
                o = 2 * LANES * g
                cc_lo = plsc.bitcast(
                    tucb[b][r, pl.ds(PSLAB + o, LANES)], jnp.float32)
                cc_hi = plsc.bitcast(
                    tucb[b][r, pl.ds(PSLAB + o + LANES, LANES)],
                    jnp.float32)
                e_lo = jnp.exp(_unpack_lo(nxf) + _unpack_lo(nhu))
                fecb[b][r, pl.ds(o, LANES)] = cc_lo / (1.0 + e_lo)
                e_hi = jnp.exp(_unpack_hi(nxf) + _unpack_hi(nhu))
                fecb[b][r, pl.ds(o + LANES, LANES)] = cc_hi / (1.0 + e_hi)

    for s_local in range(PASSES):
        slab = PASSES * core + s_local

        @pl.when(sid < ROW_TILES)
        def _zero():
            pltpu.sync_copy(zeros_hbm, acc_fc.at[pl.ds(rbase, ROWS_PER_TILE)])

        plsc.subcore_barrier()

        fire_gathers(0, 0, slab)
        fire_gathers(1, 1, slab)

        def blockpair(g2, carry):
            for b in range(2):
                i = 2 * g2 + b
                wait_gathers(b)

                @pl.when(i >= 2)
                def _ws():
                    wait_fec_scatter(i, b)

                compute(b)
                pltpu.async_copy(fecb[b], acc_fc.at[idxp.at[i]], ssem[b],
                                 add=True)

                @pl.when(i + 2 < NBLK)
                def _fg():
                    fire_gathers(i + 2, b, slab)

            return carry

        lax.fori_loop(0, NBLK // 2, blockpair, 0)
        i_last = NBLK - 1
        wait_gathers(0)
        wait_fec_scatter(i_last, 0)
        compute(0)
        pltpu.async_copy(fecb[0], acc_fc.at[idxp.at[i_last]], ssem[0],
                         add=True)
        wait_fec_scatter(i_last, 0)
        wait_fec_scatter(i_last, 1)
        plsc.subcore_barrier()

        @pl.when(sid < ROW_TILES)
        def _copy_out():
            obase = slab * N + rbase
            pltpu.sync_copy(acc_fc.at[pl.ds(rbase, ROWS_PER_TILE)],
                            fc_hbm.at[pl.ds(obase, ROWS_PER_TILE)])

        plsc.subcore_barrier()


_sc_fec = functools.partial(
    pl.kernel,
    out_type=jax.ShapeDtypeStruct((NSLAB * N, SLAB), jnp.float32),
    mesh=plsc.VectorSubcoreMesh(
        core_axis_name="c", subcore_axis_name="s",
        num_cores=NC, num_subcores=NS),
    compiler_params=pltpu.CompilerParams(use_tc_tiling_on_sc=False, needs_layout_passes=False),
    scratch_types=[
        pltpu.VMEM((NBLK, EB), jnp.int32),
        pltpu.VMEM((NBLK, EB), jnp.int32),
        pltpu.VMEM((EB, PSLAB + SLAB), jnp.int32),
        pltpu.VMEM((EB, PSLAB + SLAB), jnp.int32),
        pltpu.VMEM((EB, PSLAB), jnp.int32),
        pltpu.VMEM((EB, PSLAB), jnp.int32),
        pltpu.VMEM((EB, SLAB), jnp.float32),
        pltpu.VMEM((EB, SLAB), jnp.float32),
        pltpu.VMEM_SHARED((N, SLAB), jnp.float32),
        pltpu.SemaphoreType.DMA,
        pltpu.SemaphoreType.DMA,
        pltpu.SemaphoreType.DMA,
        pltpu.SemaphoreType.DMA,
    ],
)(_sc_fec_body)


def _level_body(hs_ref, fc_ref, xiou_ref, u_ref, uf_ref,
                th_ref, tuc_ref, h_ref, c_ref):
    hs = jnp.concatenate([hs_ref[s] for s in range(NSLAB)], axis=1)
    uz = jnp.dot(hs, u_ref[...], preferred_element_type=jnp.float32)
    xiou = xiou_ref[...]
    i_g = jax.nn.sigmoid(xiou[:, :H] + uz[:, :H])
    o_g = jax.nn.sigmoid(xiou[:, H:2 * H] + uz[:, H:2 * H])
    u_g = jnp.tanh(xiou[:, 2 * H:] + uz[:, 2 * H:])
    fc = jnp.concatenate([fc_ref[s] for s in range(NSLAB)], axis=1)
    c_new = i_g * u_g + fc
    h_new = o_g * jnp.tanh(c_new)
    hu = jnp.dot(h_new, uf_ref[...], preferred_element_type=jnp.float32)
    for s in range(NSLAB):
        sl = slice(s * SLAB, (s + 1) * SLAB)
        th_ref[s] = _pack_rows(h_new[:, sl])
        tuc_ref[s, :, :PSLAB] = _pack_rows(-hu[:, sl])
        tuc_ref[s, :, PSLAB:] = jax.lax.bitcast_convert_type(
            c_new[:, sl], jnp.int32)
    h_ref[...] = h_new
    c_ref[...] = c_new


def _level_call(hs, fc, xiou, U, U_f):
    return pl.pallas_call(
        _level_body,
        grid=(GRID_TC,),
        in_specs=[
            pl.BlockSpec((NSLAB, NBLOCK_TC, SLAB), lambda i: (0, i, 0)),
            pl.BlockSpec((NSLAB, NBLOCK_TC, SLAB), lambda i: (0, i, 0)),
            pl.BlockSpec((NBLOCK_TC, 3 * H), lambda i: (i, 0)),
            pl.BlockSpec((H, 3 * H), lambda i: (0, 0)),
            pl.BlockSpec((H, H), lambda i: (0, 0)),
        ],
        out_specs=[
            pl.BlockSpec((NSLAB, NBLOCK_TC, PSLAB), lambda i: (0, i, 0)),
            pl.BlockSpec((NSLAB, NBLOCK_TC, PSLAB + SLAB), lambda i: (0, i, 0)),
            pl.BlockSpec((NBLOCK_TC, H), lambda i: (i, 0)),
            pl.BlockSpec((NBLOCK_TC, H), lambda i: (i, 0)),
        ],
        out_shape=[
            jax.ShapeDtypeStruct((NSLAB, N, PSLAB), jnp.int32),
            jax.ShapeDtypeStruct((NSLAB, N, PSLAB + SLAB), jnp.int32),
            jax.ShapeDtypeStruct((N, H), jnp.float32),
            jax.ShapeDtypeStruct((N, H), jnp.float32),
        ],
    )(hs, fc, xiou, U, U_f)


def kernel(x, edge_index, h0, c0, W, U, U_f, b, b_f):
    child = edge_index[0].reshape(E // EB, EB)
    parent = edge_index[1].reshape(E // EB, EB)
    b2 = b.reshape(1, 4 * H)
    bf2 = b_f.reshape(1, H)
    xiou, xf2, th, tuc = _prep_call(x, W, b2, bf2, h0, c0, U_f)
    zeros = jnp.zeros((ROWS_PER_TILE, SLAB), jnp.float32)
    h = c = None
    for _ in range(LEVELS):
        hsum = _sc_hsum(th, child, parent, zeros)
        fcv = _sc_fec(tuc, xf2, child, parent, zeros)
        th, tuc, h, c = _level_call(
            hsum.reshape(NSLAB, N, SLAB),
            fcv.reshape(NSLAB, N, SLAB),
            xiou, U, U_f)
    return h, c
